# baseline (speedup 1.0000x reference)
import functools

import jax
import jax.numpy as jnp
from jax import lax
from jax.experimental import pallas as pl
from jax.experimental.pallas import tpu as pltpu
from jax.experimental.pallas import tpu_sc as plsc

N = 131072
NC, NS, L = 2, 16, 16
NW = NC * NS
CHUNK = N // NW
B = 1024
W = 2 * B + L


def _sc_body(yhat_hbm, y_hbm, out_hbm, yh_v, y_v, bins_v, sem1, sem2):
    wid = lax.axis_index("s") * NC + lax.axis_index("c")
    base = wid * CHUNK
    cp1 = pltpu.async_copy(yhat_hbm.at[pl.ds(base, CHUNK)], yh_v, sem1)
    cp2 = pltpu.async_copy(y_hbm.at[pl.ds(base, CHUNK)], y_v, sem2)

    zero16 = jnp.zeros((L,), jnp.float32)
    one16 = jnp.ones((L,), jnp.float32)

    def zbody(i, carry):
        bins_v[pl.ds(i * L, L)] = zero16
        return carry

    lax.fori_loop(0, 2 * B // L, zbody, 0)
    cp1.wait()
    cp2.wait()

    coff = jnp.full((L,), B, jnp.int32)

    UNROLL = 4

    def body(i, l1):
        for k in range(UNROLL):
            j = (UNROLL * i + k) * L
            yh = yh_v[pl.ds(j, L)]
            y = y_v[pl.ds(j, L)]
            plsc.addupdate_scatter(bins_v, [y], jnp.exp(yh))
            plsc.addupdate_scatter(bins_v, [y + coff], one16)
            l1 = l1 + jnp.where(y > 0, yh, zero16)
        return l1

    l1 = lax.fori_loop(0, CHUNK // (UNROLL * L), body, zero16)
    bins_v[pl.ds(2 * B, L)] = l1

    pltpu.sync_copy(bins_v, out_hbm.at[wid])


_sc_call = functools.partial(
    pl.kernel,
    out_type=[jax.ShapeDtypeStruct((NW, W), jnp.float32)],
    mesh=plsc.VectorSubcoreMesh(
        core_axis_name="c", subcore_axis_name="s", num_cores=NC,
        num_subcores=NS),
    scratch_types=[
        pltpu.VMEM((CHUNK,), jnp.float32),
        pltpu.VMEM((CHUNK,), jnp.int32),
        pltpu.VMEM((W,), jnp.float32),
        pltpu.SemaphoreType.DMA,
        pltpu.SemaphoreType.DMA,
    ],
    compiler_params=pltpu.CompilerParams(needs_layout_passes=False),
)(_sc_body)


def _tc_body(p_ref, out_ref):
    s8 = jnp.sum(p_ref[:, :B], axis=0).reshape(8, 128)
    c8 = jnp.sum(p_ref[:, B:2 * B], axis=0).reshape(8, 128)
    v0 = (lax.broadcasted_iota(jnp.int32, (8, 128), 0) +
          lax.broadcasted_iota(jnp.int32, (8, 128), 1)) > 0
    c8 = jnp.where(v0, c8, 0.0)
    li = lax.broadcasted_iota(jnp.int32, (128, 128), 0)
    lj = lax.broadcasted_iota(jnp.int32, (128, 128), 1)
    tri = (li <= lj).astype(jnp.float32)
    lanecum = jnp.dot(s8, tri, preferred_element_type=jnp.float32)
    totb = jnp.dot(s8, (li == li).astype(jnp.float32),
                   preferred_element_type=jnp.float32)
    ri = lax.broadcasted_iota(jnp.int32, (8, 8), 0)
    rj = lax.broadcasted_iota(jnp.int32, (8, 8), 1)
    stri = (rj < ri).astype(jnp.float32)
    rowcum = jnp.dot(stri, totb, preferred_element_type=jnp.float32)
    s_cum = lanecum + rowcum
    pos = c8 > 0.0
    loss2 = jnp.sum(jnp.where(pos, c8 * jnp.log(jnp.where(pos, s_cum, 1.0)),
                              0.0))
    loss1 = jnp.sum(p_ref[:, 2 * B:])
    obs = jnp.sum(c8)
    out_ref[0, 0] = (loss2 - loss1) / obs


_tc_call = pl.pallas_call(
    _tc_body,
    out_specs=pl.BlockSpec(memory_space=pltpu.MemorySpace.SMEM),
    out_shape=jax.ShapeDtypeStruct((1, 1), jnp.float32),
)


def kernel(Yhat, Y):
    Yhat = jnp.squeeze(Yhat)
    Y = jnp.squeeze(Y)
    (part,) = _sc_call(Yhat, Y)
    out = _tc_call(part)
    return out[0, 0]


# speedup vs baseline: 1.0089x; 1.0089x over previous
import functools

import jax
import jax.numpy as jnp
from jax import lax
from jax.experimental import pallas as pl
from jax.experimental.pallas import tpu as pltpu
from jax.experimental.pallas import tpu_sc as plsc

N = 131072
NC, NS, L = 2, 16, 16
NW = NC * NS
CHUNK = N // NW
B = 1024
W = 2 * B + L


def _sc_body(yhat_hbm, y_hbm, out_hbm, yh_v, y_v, bins_v, sem1, sem2):
    wid = lax.axis_index("s") * NC + lax.axis_index("c")
    base = wid * CHUNK
    cp1 = pltpu.async_copy(yhat_hbm.at[pl.ds(base, CHUNK)], yh_v, sem1)
    cp2 = pltpu.async_copy(y_hbm.at[pl.ds(base, CHUNK)], y_v, sem2)

    zero16 = jnp.zeros((L,), jnp.float32)
    one16 = jnp.ones((L,), jnp.float32)

    def zbody(i, carry):
        bins_v[pl.ds(i * L, L)] = zero16
        return carry

    lax.fori_loop(0, 2 * B // L, zbody, 0)
    cp1.wait()
    cp2.wait()

    coff = jnp.full((L,), B, jnp.int32)

    UNROLL = 2

    def body(i, l1):
        for k in range(UNROLL):
            j = (UNROLL * i + k) * L
            yh = yh_v[pl.ds(j, L)]
            y = y_v[pl.ds(j, L)]
            plsc.addupdate_scatter(bins_v, [y], jnp.exp(yh))
            plsc.addupdate_scatter(bins_v, [y + coff], one16)
            l1 = l1 + jnp.where(y > 0, yh, zero16)
        return l1

    l1 = lax.fori_loop(0, CHUNK // (UNROLL * L), body, zero16)
    bins_v[pl.ds(2 * B, L)] = l1

    pltpu.sync_copy(bins_v, out_hbm.at[wid])


_sc_call = functools.partial(
    pl.kernel,
    out_type=[jax.ShapeDtypeStruct((NW, W), jnp.float32)],
    mesh=plsc.VectorSubcoreMesh(
        core_axis_name="c", subcore_axis_name="s", num_cores=NC,
        num_subcores=NS),
    scratch_types=[
        pltpu.VMEM((CHUNK,), jnp.float32),
        pltpu.VMEM((CHUNK,), jnp.int32),
        pltpu.VMEM((W,), jnp.float32),
        pltpu.SemaphoreType.DMA,
        pltpu.SemaphoreType.DMA,
    ],
    compiler_params=pltpu.CompilerParams(needs_layout_passes=False),
)(_sc_body)


def _tc_body(p_ref, out_ref):
    s8 = jnp.sum(p_ref[:, :B], axis=0).reshape(8, 128)
    c8 = jnp.sum(p_ref[:, B:2 * B], axis=0).reshape(8, 128)
    v0 = (lax.broadcasted_iota(jnp.int32, (8, 128), 0) +
          lax.broadcasted_iota(jnp.int32, (8, 128), 1)) > 0
    c8 = jnp.where(v0, c8, 0.0)
    li = lax.broadcasted_iota(jnp.int32, (128, 128), 0)
    lj = lax.broadcasted_iota(jnp.int32, (128, 128), 1)
    tri = (li <= lj).astype(jnp.float32)
    lanecum = jnp.dot(s8, tri, preferred_element_type=jnp.float32)
    totb = jnp.dot(s8, (li == li).astype(jnp.float32),
                   preferred_element_type=jnp.float32)
    ri = lax.broadcasted_iota(jnp.int32, (8, 8), 0)
    rj = lax.broadcasted_iota(jnp.int32, (8, 8), 1)
    stri = (rj < ri).astype(jnp.float32)
    rowcum = jnp.dot(stri, totb, preferred_element_type=jnp.float32)
    s_cum = lanecum + rowcum
    pos = c8 > 0.0
    loss2 = jnp.sum(jnp.where(pos, c8 * jnp.log(jnp.where(pos, s_cum, 1.0)),
                              0.0))
    loss1 = jnp.sum(p_ref[:, 2 * B:])
    obs = jnp.sum(c8)
    out_ref[0, 0] = (loss2 - loss1) / obs


_tc_call = pl.pallas_call(
    _tc_body,
    out_specs=pl.BlockSpec(memory_space=pltpu.MemorySpace.SMEM),
    out_shape=jax.ShapeDtypeStruct((1, 1), jnp.float32),
)


def kernel(Yhat, Y):
    Yhat = jnp.squeeze(Yhat)
    Y = jnp.squeeze(Y)
    (part,) = _sc_call(Yhat, Y)
    out = _tc_call(part)
    return out[0, 0]
